# Initial kernel scaffold; baseline (speedup 1.0000x reference)
import jax
import jax.numpy as jnp
from jax.experimental import pallas as pl


def kernel(input_ids, embedding_table, hard_norm_scale, projection_weight):
    raise NotImplementedError("write your pallas kernel here")


# speedup vs baseline: 1.7501x; 1.7501x over previous
import jax
import jax.numpy as jnp
from jax.experimental import pallas as pl
from jax.experimental.pallas import tpu as pltpu
from jax.experimental.pallas import tpu_sc as plsc

MM_HIDDEN = 128
TEXT_HIDDEN = 2048
EPS = 1e-06

GATHER_WINDOW = 256
ROW_BLOCK = 1024


def _sc_gather(table, ids_flat):
    n = ids_flat.shape[0]
    ids2d = ids_flat.reshape(1, n)
    mesh = plsc.VectorSubcoreMesh(core_axis_name="core", subcore_axis_name="subcore")

    @pl.kernel(
        out_type=jax.ShapeDtypeStruct((n, MM_HIDDEN), table.dtype),
        mesh=mesh,
    )
    def gather_kernel(table_hbm, ids_hbm, out_hbm):
        def body(i_vmem, o_vmem):
            pltpu.sync_copy(table_hbm.at[i_vmem.at[0]], o_vmem)

        pltpu.emit_pipeline(
            body,
            grid=(n // GATHER_WINDOW,),
            in_specs=[pl.BlockSpec((1, GATHER_WINDOW), lambda i: (0, i))],
            out_specs=[pl.BlockSpec((GATHER_WINDOW, MM_HIDDEN), lambda i: (i, 0))],
            core_axis_name=("core", "subcore"),
            dimension_semantics=(pltpu.PARALLEL,),
        )(ids_hbm, out_hbm)

    return gather_kernel(table, ids2d)


def _tc_body(x_ref, s_ref, w_ref, o_ref):
    x = x_ref[...]
    inv1 = jax.lax.rsqrt(jnp.mean(x * x, axis=1, keepdims=True) + EPS)
    y = x * inv1 * s_ref[...]
    z = jax.lax.dot_general(
        y,
        w_ref[...],
        (((1,), (0,)), ((), ())),
        precision=jax.lax.Precision.HIGHEST,
        preferred_element_type=jnp.float32,
    )
    inv2 = jax.lax.rsqrt(jnp.mean(z * z, axis=1, keepdims=True) + EPS)
    o_ref[...] = z * inv2


def _tc_norm_proj_norm(gathered, scale, weight):
    n = gathered.shape[0]
    return pl.pallas_call(
        _tc_body,
        grid=(n // ROW_BLOCK,),
        in_specs=[
            pl.BlockSpec((ROW_BLOCK, MM_HIDDEN), lambda i: (i, 0)),
            pl.BlockSpec((1, MM_HIDDEN), lambda i: (0, 0)),
            pl.BlockSpec((MM_HIDDEN, TEXT_HIDDEN), lambda i: (0, 0)),
        ],
        out_specs=pl.BlockSpec((ROW_BLOCK, TEXT_HIDDEN), lambda i: (i, 0)),
        out_shape=jax.ShapeDtypeStruct((n, TEXT_HIDDEN), jnp.float32),
        compiler_params=pltpu.CompilerParams(
            dimension_semantics=("parallel",),
        ),
    )(gathered, scale.reshape(1, MM_HIDDEN), weight)


def kernel(input_ids, embedding_table, hard_norm_scale, projection_weight):
    b, s = input_ids.shape
    ids_flat = input_ids.reshape(b * s)
    gathered = _sc_gather(embedding_table, ids_flat)
    out = _tc_norm_proj_norm(gathered, hard_norm_scale, projection_weight)
    return out.reshape(b, s, TEXT_HIDDEN)


# speedup vs baseline: 3.5160x; 2.0091x over previous
import jax
import jax.numpy as jnp
from jax.experimental import pallas as pl
from jax.experimental.pallas import tpu as pltpu
from jax.experimental.pallas import tpu_sc as plsc

MM_HIDDEN = 128
TEXT_HIDDEN = 2048
EPS = 1e-06

GATHER_WINDOW = 256
ROW_BLOCK = 1024


def _sc_gather(table, ids_flat):
    n = ids_flat.shape[0]
    ids2d = ids_flat.reshape(1, n)
    mesh = plsc.VectorSubcoreMesh(core_axis_name="core", subcore_axis_name="subcore")

    @pl.kernel(
        out_type=jax.ShapeDtypeStruct((n, MM_HIDDEN), table.dtype),
        mesh=mesh,
    )
    def gather_kernel(table_hbm, ids_hbm, out_hbm):
        def body(i_vmem, o_vmem):
            pltpu.sync_copy(table_hbm.at[i_vmem.at[0]], o_vmem)

        pltpu.emit_pipeline(
            body,
            grid=(n // GATHER_WINDOW,),
            in_specs=[pl.BlockSpec((1, GATHER_WINDOW), lambda i: (0, i))],
            out_specs=[pl.BlockSpec((GATHER_WINDOW, MM_HIDDEN), lambda i: (i, 0))],
            core_axis_name=("core", "subcore"),
            dimension_semantics=(pltpu.PARALLEL,),
        )(ids_hbm, out_hbm)

    return gather_kernel(table, ids2d)


def _tc_body(x_ref, s_ref, w_ref, o_ref):
    x = x_ref[...]
    inv1 = jax.lax.rsqrt(jnp.mean(x * x, axis=1, keepdims=True) + EPS)
    y = (x * inv1 * s_ref[...]).astype(jnp.bfloat16)
    z = jax.lax.dot_general(
        y,
        w_ref[...].astype(jnp.bfloat16),
        (((1,), (0,)), ((), ())),
        preferred_element_type=jnp.float32,
    )
    inv2 = jax.lax.rsqrt(jnp.mean(z * z, axis=1, keepdims=True) + EPS)
    o_ref[...] = z * inv2


def _tc_norm_proj_norm(gathered, scale, weight):
    n = gathered.shape[0]
    return pl.pallas_call(
        _tc_body,
        grid=(n // ROW_BLOCK,),
        in_specs=[
            pl.BlockSpec((ROW_BLOCK, MM_HIDDEN), lambda i: (i, 0)),
            pl.BlockSpec((1, MM_HIDDEN), lambda i: (0, 0)),
            pl.BlockSpec((MM_HIDDEN, TEXT_HIDDEN), lambda i: (0, 0)),
        ],
        out_specs=pl.BlockSpec((ROW_BLOCK, TEXT_HIDDEN), lambda i: (i, 0)),
        out_shape=jax.ShapeDtypeStruct((n, TEXT_HIDDEN), jnp.float32),
        compiler_params=pltpu.CompilerParams(
            dimension_semantics=("parallel",),
        ),
    )(gathered, scale.reshape(1, MM_HIDDEN), weight)


def kernel(input_ids, embedding_table, hard_norm_scale, projection_weight):
    b, s = input_ids.shape
    ids_flat = input_ids.reshape(b * s)
    gathered = _sc_gather(embedding_table, ids_flat)
    out = _tc_norm_proj_norm(gathered, hard_norm_scale, projection_weight)
    return out.reshape(b, s, TEXT_HIDDEN)


# speedup vs baseline: 3.5606x; 1.0127x over previous
import jax
import jax.numpy as jnp
from jax.experimental import pallas as pl
from jax.experimental.pallas import tpu as pltpu
from jax.experimental.pallas import tpu_sc as plsc

MM_HIDDEN = 128
TEXT_HIDDEN = 2048
EPS = 1e-06

GATHER_WINDOW = 256
ROW_BLOCK = 1024


def _sc_gather(table, ids_flat):
    n = ids_flat.shape[0]
    ids2d = ids_flat.reshape(1, n)
    mesh = plsc.VectorSubcoreMesh(core_axis_name="core", subcore_axis_name="subcore")

    @pl.kernel(
        out_type=jax.ShapeDtypeStruct((n, MM_HIDDEN), table.dtype),
        mesh=mesh,
    )
    def gather_kernel(table_hbm, ids_hbm, out_hbm):
        def body(i_vmem, o_vmem):
            pltpu.sync_copy(table_hbm.at[i_vmem.at[0]], o_vmem)

        pltpu.emit_pipeline(
            body,
            grid=(n // GATHER_WINDOW,),
            in_specs=[pl.BlockSpec((1, GATHER_WINDOW), lambda i: (0, i))],
            out_specs=[pl.BlockSpec((GATHER_WINDOW, MM_HIDDEN), lambda i: (i, 0))],
            core_axis_name=("core", "subcore"),
            dimension_semantics=(pltpu.PARALLEL,),
        )(ids_hbm, out_hbm)

    return gather_kernel(table, ids2d)


def _tc_body(x_ref, s_ref, w_ref, o_ref, w16_ref, g16_ref):
    @pl.when(pl.program_id(0) == 0)
    def _():
        w16 = w_ref[...].astype(jnp.bfloat16)
        w16_ref[...] = w16
        g = jax.lax.dot_general(
            w16, w16, (((1,), (1,)), ((), ())),
            preferred_element_type=jnp.float32,
        )
        g16_ref[...] = g.astype(jnp.bfloat16)

    x = x_ref[...]
    inv1 = jax.lax.rsqrt(jnp.mean(x * x, axis=1, keepdims=True) + EPS)
    y32 = x * inv1 * s_ref[...]
    y = y32.astype(jnp.bfloat16)
    t = jax.lax.dot_general(
        y, g16_ref[...], (((1,), (0,)), ((), ())),
        preferred_element_type=jnp.float32,
    )
    q = jnp.sum(t * y32, axis=1, keepdims=True)
    inv2 = jax.lax.rsqrt(q / TEXT_HIDDEN + EPS)
    z = jax.lax.dot_general(
        y, w16_ref[...], (((1,), (0,)), ((), ())),
        preferred_element_type=jnp.float32,
    )
    o_ref[...] = z * inv2


def _tc_norm_proj_norm(gathered, scale, weight):
    n = gathered.shape[0]
    return pl.pallas_call(
        _tc_body,
        grid=(n // ROW_BLOCK,),
        in_specs=[
            pl.BlockSpec((ROW_BLOCK, MM_HIDDEN), lambda i: (i, 0)),
            pl.BlockSpec((1, MM_HIDDEN), lambda i: (0, 0)),
            pl.BlockSpec((MM_HIDDEN, TEXT_HIDDEN), lambda i: (0, 0)),
        ],
        out_specs=pl.BlockSpec((ROW_BLOCK, TEXT_HIDDEN), lambda i: (i, 0)),
        out_shape=jax.ShapeDtypeStruct((n, TEXT_HIDDEN), jnp.float32),
        scratch_shapes=[
            pltpu.VMEM((MM_HIDDEN, TEXT_HIDDEN), jnp.bfloat16),
            pltpu.VMEM((MM_HIDDEN, MM_HIDDEN), jnp.bfloat16),
        ],
        compiler_params=pltpu.CompilerParams(
            dimension_semantics=("arbitrary",),
        ),
    )(gathered, scale.reshape(1, MM_HIDDEN), weight)


def kernel(input_ids, embedding_table, hard_norm_scale, projection_weight):
    b, s = input_ids.shape
    ids_flat = input_ids.reshape(b * s)
    gathered = _sc_gather(embedding_table, ids_flat)
    out = _tc_norm_proj_norm(gathered, hard_norm_scale, projection_weight)
    return out.reshape(b, s, TEXT_HIDDEN)
